# baseline (speedup 1.0000x reference)
import functools

import jax
import jax.numpy as jnp
from jax import lax
from jax.experimental import pallas as pl
from jax.experimental.pallas import tpu as pltpu
from jax.experimental.pallas import tpu_sc as plsc

N = 320000
D = 128
NSEG = 10000
NSEG_PAD = 10240

NC = 2
NS = 16
L = 16

SCAN = N // NS
OUT = N // (NC * NS)
COLS = NSEG_PAD // NS

TC_ROWS = 16000


def _tc_logit_body(feat_ref, g_ref, out_ref):
    i = pl.program_id(0)
    x = feat_ref[...]
    g = g_ref[...]
    z = lax.dot_general(g, x, (((1,), (1,)), ((), ())),
                        preferred_element_type=jnp.float32)
    out_ref[pl.ds(i * TC_ROWS, TC_ROWS)] = jnp.exp(z).reshape(TC_ROWS)


def _compute_logit(features, gamma_w):
    grid = N // TC_ROWS
    return pl.pallas_call(
        _tc_logit_body,
        grid=(grid,),
        in_specs=[
            pl.BlockSpec((TC_ROWS, D), lambda i: (i, 0)),
            pl.BlockSpec((1, D), lambda i: (0, 0)),
        ],
        out_specs=pl.BlockSpec((N,), lambda i: (0,)),
        out_shape=jax.ShapeDtypeStruct((N,), jnp.float32),
    )(features, gamma_w)


def _sc_seg_body(logit_hbm, pid_hbm, out_hbm,
                 ids_buf, lg_buf, box, tmp, acc, out_buf,
                 shared_all, shared_gbox, sem1, sem2):
    s = lax.axis_index("s")
    cid = lax.axis_index("c")
    scan_base = s * SCAN
    off = cid * OUT
    out_base = scan_base + off

    cp1 = pltpu.async_copy(logit_hbm.at[pl.ds(scan_base, SCAN)], lg_buf, sem1)
    cp2 = pltpu.async_copy(pid_hbm.at[pl.ds(scan_base, SCAN)],
                           ids_buf.at[pl.ds(L, SCAN)], sem2)

    zero = jnp.zeros((L,), jnp.float32)

    @plsc.parallel_loop(0, NSEG_PAD // L, unroll=8)
    def zero_body(i):
        box[pl.ds(i * L, L)] = zero

    cp1.wait()
    cp2.wait()

    lane = lax.iota(jnp.int32, L)
    is_first = lane == 0
    is_last = lane == L - 1

    with jax.named_scope("sc_scan"):
        @plsc.parallel_loop(0, SCAN // L, unroll=8)
        def scan_body(i):
            base = i * L
            v = lg_buf[pl.ds(base, L)]
            ids = ids_buf[pl.ds(L + base, L)]
            ids_prev = ids_buf[pl.ds(L - 1 + base, L)]
            ids_next = ids_buf[pl.ds(L + 1 + base, L)]
            c = plsc.cumsum(v)
            end_m = (ids != ids_next) | is_last
            start_m = (ids != ids_prev) | is_first
            plsc.addupdate_scatter(box, [ids], c, mask=end_m)
            plsc.addupdate_scatter(box, [ids], v - c, mask=start_m)

    with jax.named_scope("sc_combine"):
        pltpu.sync_copy(box, shared_all.at[s])
        plsc.subcore_barrier()

        col0 = s * COLS
        pltpu.sync_copy(shared_all.at[:, pl.ds(col0, COLS)], tmp)

        one = jnp.ones((L,), jnp.float32)

        @plsc.parallel_loop(0, COLS // L, unroll=2)
        def red_body(j):
            jb = j * L
            vec = tmp[0, pl.ds(jb, L)]
            for r in range(1, NS):
                vec = vec + tmp[r, pl.ds(jb, L)]
            acc[pl.ds(jb, L)] = one / vec
        pltpu.sync_copy(acc, shared_gbox.at[pl.ds(col0, COLS)])
        plsc.subcore_barrier()

        pltpu.sync_copy(shared_gbox, box)

    with jax.named_scope("sc_norm"):
        @plsc.parallel_loop(0, OUT // L, unroll=4)
        def norm_body(i):
            base = off + i * L
            ids = ids_buf[pl.ds(L + base, L)]
            v = lg_buf[pl.ds(base, L)]
            g = plsc.load_gather(box, [ids])
            out_buf[pl.ds(i * L, L)] = v * g
        pltpu.sync_copy(out_buf, out_hbm.at[pl.ds(out_base, OUT)])


@jax.jit
def kernel(features, phrase_id, unique_phrase, gamma_w):
    logit = _compute_logit(features, gamma_w)
    mesh = plsc.VectorSubcoreMesh(
        core_axis_name="c", subcore_axis_name="s",
        num_cores=NC, num_subcores=NS)
    sc_kernel = functools.partial(
        pl.kernel,
        out_type=jax.ShapeDtypeStruct((N,), jnp.float32),
        mesh=mesh,
        compiler_params=pltpu.CompilerParams(needs_layout_passes=False),
        scratch_types=[
            pltpu.VMEM((SCAN + 2 * L,), jnp.int32),
            pltpu.VMEM((SCAN,), jnp.float32),
            pltpu.VMEM((NSEG_PAD,), jnp.float32),
            pltpu.VMEM((NS, COLS), jnp.float32),
            pltpu.VMEM((COLS,), jnp.float32),
            pltpu.VMEM((OUT,), jnp.float32),
            pltpu.VMEM_SHARED((NS, NSEG_PAD), jnp.float32),
            pltpu.VMEM_SHARED((NSEG_PAD,), jnp.float32),
            pltpu.SemaphoreType.DMA,
            pltpu.SemaphoreType.DMA,
        ],
    )(_sc_seg_body)
    weights = sc_kernel(logit, phrase_id)
    return weights[:, None]


# speedup vs baseline: 1.0168x; 1.0168x over previous
import functools

import jax
import jax.numpy as jnp
from jax import lax
from jax.experimental import pallas as pl
from jax.experimental.pallas import tpu as pltpu
from jax.experimental.pallas import tpu_sc as plsc

N = 320000
D = 128
NSEG = 10000
NSEG_PAD = 10240

NC = 2
NS = 16
L = 16

SCAN = N // NS
OUT = N // (NC * NS)
COLS = NSEG_PAD // NS

TC_ROWS = 16000
PIECES = 5
PIECE = SCAN // PIECES


def _tc_logit_body(feat_ref, g_ref, out_ref):
    i = pl.program_id(0)
    x = feat_ref[...]
    g = g_ref[...]
    z = lax.dot_general(g, x, (((1,), (1,)), ((), ())),
                        preferred_element_type=jnp.float32)
    out_ref[pl.ds(i * TC_ROWS, TC_ROWS)] = jnp.exp(z).reshape(TC_ROWS)


def _compute_logit(features, gamma_w):
    grid = N // TC_ROWS
    return pl.pallas_call(
        _tc_logit_body,
        grid=(grid,),
        in_specs=[
            pl.BlockSpec((TC_ROWS, D), lambda i: (i, 0)),
            pl.BlockSpec((1, D), lambda i: (0, 0)),
        ],
        out_specs=pl.BlockSpec((N,), lambda i: (0,)),
        out_shape=jax.ShapeDtypeStruct((N,), jnp.float32),
    )(features, gamma_w)


def _sc_seg_body(logit_hbm, pid_hbm, out_hbm,
                 ids_buf, lg_buf, box, tmp, acc, out_buf,
                 shared_all, shared_gbox, sems_lg, sems_id):
    s = lax.axis_index("s")
    cid = lax.axis_index("c")
    scan_base = s * SCAN
    off = cid * OUT
    out_base = scan_base + off

    cps = []
    for p in range(PIECES):
        pb = p * PIECE
        cps.append((
            pltpu.async_copy(logit_hbm.at[pl.ds(scan_base + pb, PIECE)],
                             lg_buf.at[pl.ds(pb, PIECE)], sems_lg[p]),
            pltpu.async_copy(pid_hbm.at[pl.ds(scan_base + pb, PIECE)],
                             ids_buf.at[pl.ds(L + pb, PIECE)], sems_id[p]),
        ))

    zero = jnp.zeros((L,), jnp.float32)

    @plsc.parallel_loop(0, NSEG_PAD // L, unroll=8)
    def zero_body(i):
        box[pl.ds(i * L, L)] = zero

    lane = lax.iota(jnp.int32, L)
    is_first = lane == 0
    is_last = lane == L - 1

    with jax.named_scope("sc_scan"):
        for p in range(PIECES):
            cps[p][0].wait()
            cps[p][1].wait()
            pb = p * PIECE

            @plsc.parallel_loop(pb // L, (pb + PIECE) // L, unroll=8)
            def scan_body(i):
                base = i * L
                v = lg_buf[pl.ds(base, L)]
                ids = ids_buf[pl.ds(L + base, L)]
                ids_prev = ids_buf[pl.ds(L - 1 + base, L)]
                ids_next = ids_buf[pl.ds(L + 1 + base, L)]
                c = plsc.cumsum(v)
                end_m = (ids != ids_next) | is_last
                start_m = (ids != ids_prev) | is_first
                plsc.addupdate_scatter(box, [ids], c, mask=end_m)
                plsc.addupdate_scatter(box, [ids], v - c, mask=start_m)

    with jax.named_scope("sc_combine"):
        pltpu.sync_copy(box, shared_all.at[s])
        plsc.subcore_barrier()

        col0 = s * COLS
        pltpu.sync_copy(shared_all.at[:, pl.ds(col0, COLS)], tmp)

        one = jnp.ones((L,), jnp.float32)

        @plsc.parallel_loop(0, COLS // L, unroll=2)
        def red_body(j):
            jb = j * L
            vec = tmp[0, pl.ds(jb, L)]
            for r in range(1, NS):
                vec = vec + tmp[r, pl.ds(jb, L)]
            acc[pl.ds(jb, L)] = one / vec
        pltpu.sync_copy(acc, shared_gbox.at[pl.ds(col0, COLS)])
        plsc.subcore_barrier()

        pltpu.sync_copy(shared_gbox, box)

    with jax.named_scope("sc_norm"):
        @plsc.parallel_loop(0, OUT // L, unroll=4)
        def norm_body(i):
            base = off + i * L
            ids = ids_buf[pl.ds(L + base, L)]
            v = lg_buf[pl.ds(base, L)]
            g = plsc.load_gather(box, [ids])
            out_buf[pl.ds(i * L, L)] = v * g
        pltpu.sync_copy(out_buf, out_hbm.at[pl.ds(out_base, OUT)])


@jax.jit
def kernel(features, phrase_id, unique_phrase, gamma_w):
    logit = _compute_logit(features, gamma_w)
    mesh = plsc.VectorSubcoreMesh(
        core_axis_name="c", subcore_axis_name="s",
        num_cores=NC, num_subcores=NS)
    sc_kernel = functools.partial(
        pl.kernel,
        out_type=jax.ShapeDtypeStruct((N,), jnp.float32),
        mesh=mesh,
        compiler_params=pltpu.CompilerParams(needs_layout_passes=False),
        scratch_types=[
            pltpu.VMEM((SCAN + 2 * L,), jnp.int32),
            pltpu.VMEM((SCAN,), jnp.float32),
            pltpu.VMEM((NSEG_PAD,), jnp.float32),
            pltpu.VMEM((NS, COLS), jnp.float32),
            pltpu.VMEM((COLS,), jnp.float32),
            pltpu.VMEM((OUT,), jnp.float32),
            pltpu.VMEM_SHARED((NS, NSEG_PAD), jnp.float32),
            pltpu.VMEM_SHARED((NSEG_PAD,), jnp.float32),
            [pltpu.SemaphoreType.DMA] * PIECES,
            [pltpu.SemaphoreType.DMA] * PIECES,
        ],
    )(_sc_seg_body)
    weights = sc_kernel(logit, phrase_id)
    return weights[:, None]
